# baseline (speedup 1.0000x reference)
import functools

import jax
import jax.numpy as jnp
from jax import lax
from jax.experimental import pallas as pl
from jax.experimental.pallas import tpu as pltpu
from jax.experimental.pallas import tpu_sc as plsc

_RING = 3


def kernel(x, pos_table):
    B, S, D = x.shape
    NC, NS = 2, 16
    NW = NC * NS
    SPW = S // NW
    P = 8
    NCHUNK = SPW // P
    CPR = D // 16
    mesh = plsc.VectorSubcoreMesh(
        core_axis_name="c", subcore_axis_name="s", num_cores=NC, num_subcores=NS
    )

    @functools.partial(
        pl.kernel,
        out_type=jax.ShapeDtypeStruct((B, S, D), jnp.float32),
        mesh=mesh,
        scratch_types=[
            pltpu.VMEM((_RING, P, D), jnp.float32),
            pltpu.VMEM((_RING, B, P, D), jnp.float32),
            pltpu.SemaphoreType.DMA((_RING,)),
            pltpu.SemaphoreType.DMA((_RING,)),
        ],
    )
    def k(x_hbm, t_hbm, o_hbm, tt, xb, insem, outsem):
        wid = lax.axis_index("s") * NC + lax.axis_index("c")
        base = wid * SPW

        def in_descs(ci, sl):
            p0 = base + ci * P
            return (
                pltpu.make_async_copy(t_hbm.at[pl.ds(p0, P)], tt.at[sl], insem.at[sl]),
                pltpu.make_async_copy(
                    x_hbm.at[:, pl.ds(p0, P)], xb.at[sl], insem.at[sl]
                ),
            )

        def out_desc(ci, sl, b):
            p0 = base + ci * P
            return pltpu.make_async_copy(
                xb.at[sl, b], o_hbm.at[b, pl.ds(p0, P)], outsem.at[sl]
            )

        for d in in_descs(0, 0):
            d.start()
        for ci in range(NCHUNK):
            sl = ci % _RING
            for d in in_descs(ci, sl):
                d.wait()
            if ci >= 2:
                for b in range(B):
                    out_desc(ci - 2, (ci + 1) % _RING, b).wait()
            if ci + 1 < NCHUNK:
                for d in in_descs(ci + 1, (ci + 1) % _RING):
                    d.start()

            for b in range(B):

                def row(r, rc):
                    @plsc.parallel_loop(0, D, step=16, unroll=8)
                    def _col(j):
                        slc = pl.ds(j, 16)
                        plsc.addupdate(xb.at[sl, b, r, slc], tt[sl, r, slc])

                    return rc

                lax.fori_loop(0, P, row, 0)
                out_desc(ci, sl, b).start()
        for ci in range(NCHUNK - 2, NCHUNK):
            for b in range(B):
                out_desc(ci, ci % _RING, b).wait()

    return k(x, pos_table)


# speedup vs baseline: 1.0531x; 1.0531x over previous
import functools

import jax
import jax.numpy as jnp
from jax import lax
from jax.experimental import pallas as pl
from jax.experimental.pallas import tpu as pltpu
from jax.experimental.pallas import tpu_sc as plsc

_RING = 3


def kernel(x, pos_table):
    B, S, D = x.shape
    NC, NS = 2, 16
    NW = NC * NS
    SPW = S // NW
    P = 8
    NCHUNK = SPW // P
    CPR = D // 16
    mesh = plsc.VectorSubcoreMesh(
        core_axis_name="c", subcore_axis_name="s", num_cores=NC, num_subcores=NS
    )

    @functools.partial(
        pl.kernel,
        out_type=jax.ShapeDtypeStruct((B, S, D), jnp.float32),
        mesh=mesh,
        scratch_types=[
            pltpu.VMEM((_RING, P, D), jnp.float32),
            pltpu.VMEM((_RING, B, P, D), jnp.float32),
            pltpu.SemaphoreType.DMA((_RING,)),
            pltpu.SemaphoreType.DMA((_RING,)),
        ],
    )
    def k(x_hbm, t_hbm, o_hbm, tt, xb, insem, outsem):
        wid = lax.axis_index("s") * NC + lax.axis_index("c")
        base = wid * SPW

        def in_descs(ci, sl):
            p0 = base + ci * P
            return (
                pltpu.make_async_copy(t_hbm.at[pl.ds(p0, P)], tt.at[sl], insem.at[sl]),
                pltpu.make_async_copy(
                    x_hbm.at[:, pl.ds(p0, P)], xb.at[sl], insem.at[sl]
                ),
            )

        def out_desc(ci, sl):
            p0 = base + ci * P
            return pltpu.make_async_copy(
                xb.at[sl], o_hbm.at[:, pl.ds(p0, P)], outsem.at[sl]
            )

        for d in in_descs(0, 0):
            d.start()
        for ci in range(NCHUNK):
            sl = ci % _RING
            for d in in_descs(ci, sl):
                d.wait()
            if ci >= 2:
                out_desc(ci - 2, (ci + 1) % _RING).wait()
            if ci + 1 < NCHUNK:
                for d in in_descs(ci + 1, (ci + 1) % _RING):
                    d.start()

            def row(r, rc):
                @plsc.parallel_loop(0, D, step=16, unroll=8)
                def _col(j):
                    slc = pl.ds(j, 16)
                    tv = tt[sl, r, slc]
                    for b in range(B):
                        plsc.addupdate(xb.at[sl, b, r, slc], tv)

                return rc

            lax.fori_loop(0, P, row, 0)
            out_desc(ci, sl).start()
        for ci in range(NCHUNK - 2, NCHUNK):
            out_desc(ci, ci % _RING).wait()

    return k(x, pos_table)


# speedup vs baseline: 1.0668x; 1.0130x over previous
import functools

import jax
import jax.numpy as jnp
from jax import lax
from jax.experimental import pallas as pl
from jax.experimental.pallas import tpu as pltpu
from jax.experimental.pallas import tpu_sc as plsc

_RING = 3


def _sc_part(x, pos_table, K):
    B, S, D = x.shape
    NC, NS = 2, 16
    NW = NC * NS
    SPW = K // NW
    P = 8
    NCHUNK = SPW // P
    mesh = plsc.VectorSubcoreMesh(
        core_axis_name="c", subcore_axis_name="s", num_cores=NC, num_subcores=NS
    )

    @functools.partial(
        pl.kernel,
        out_type=jax.ShapeDtypeStruct((B, K, D), jnp.float32),
        mesh=mesh,
        scratch_types=[
            pltpu.VMEM((_RING, P, D), jnp.float32),
            pltpu.VMEM((_RING, B, P, D), jnp.float32),
            pltpu.SemaphoreType.DMA((_RING,)),
            pltpu.SemaphoreType.DMA((_RING,)),
        ],
    )
    def k(x_hbm, t_hbm, o_hbm, tt, xb, insem, outsem):
        wid = lax.axis_index("s") * NC + lax.axis_index("c")
        base = wid * SPW

        def in_descs(ci, sl):
            p0 = base + ci * P
            return (
                pltpu.make_async_copy(t_hbm.at[pl.ds(p0, P)], tt.at[sl], insem.at[sl]),
                pltpu.make_async_copy(
                    x_hbm.at[:, pl.ds(p0, P)], xb.at[sl], insem.at[sl]
                ),
            )

        def out_desc(ci, sl):
            p0 = base + ci * P
            return pltpu.make_async_copy(
                xb.at[sl], o_hbm.at[:, pl.ds(p0, P)], outsem.at[sl]
            )

        for d in in_descs(0, 0):
            d.start()
        for ci in range(NCHUNK):
            sl = ci % _RING
            for d in in_descs(ci, sl):
                d.wait()
            if ci >= 2:
                out_desc(ci - 2, (ci + 1) % _RING).wait()
            if ci + 1 < NCHUNK:
                for d in in_descs(ci + 1, (ci + 1) % _RING):
                    d.start()

            def row(r, rc):
                @plsc.parallel_loop(0, D, step=16, unroll=8)
                def _col(j):
                    slc = pl.ds(j, 16)
                    tv = tt[sl, r, slc]
                    for b in range(B):
                        plsc.addupdate(xb.at[sl, b, r, slc], tv)

                return rc

            lax.fori_loop(0, P, row, 0)
            out_desc(ci, sl).start()
        for ci in range(max(NCHUNK - 2, 0), NCHUNK):
            out_desc(ci, ci % _RING).wait()

    return k(x, pos_table)


def _tc_add_body(x_ref, t_ref, o_ref):
    o_ref[...] = x_ref[...] + t_ref[None]


def _merge_body(s_ref, f_ref, o_ref):
    o_ref[...] = s_ref[...]


def kernel(x, pos_table):
    B, S, D = x.shape
    K = S // 4
    SBLK = 512

    sc_out = _sc_part(x, pos_table, K)

    tc_full = pl.pallas_call(
        _tc_add_body,
        grid=((S - K) // SBLK,),
        in_specs=[
            pl.BlockSpec((B, SBLK, D), lambda i: (0, i + K // SBLK, 0)),
            pl.BlockSpec((SBLK, D), lambda i: (i + K // SBLK, 0)),
        ],
        out_specs=pl.BlockSpec((B, SBLK, D), lambda i: (0, i + K // SBLK, 0)),
        out_shape=jax.ShapeDtypeStruct((B, S, D), x.dtype),
    )(x, pos_table)

    return pl.pallas_call(
        _merge_body,
        grid=(K // SBLK,),
        in_specs=[
            pl.BlockSpec((B, SBLK, D), lambda i: (0, i, 0)),
            pl.BlockSpec(memory_space=pl.ANY),
        ],
        out_specs=pl.BlockSpec((B, SBLK, D), lambda i: (0, i, 0)),
        out_shape=jax.ShapeDtypeStruct((B, S, D), x.dtype),
        input_output_aliases={1: 0},
    )(sc_out, tc_full)
